# Initial kernel scaffold; baseline (speedup 1.0000x reference)
import functools

import jax
import jax.numpy as jnp
from jax import lax
from jax.experimental import pallas as pl
from jax.experimental.pallas import tpu as pltpu
from jax.experimental.pallas import tpu_sc as plsc

N = 10000
E = 320000
HID = 128
HEADS = 8
DIM = 8
DI = HEADS * DIM
CLAMP = 5.0

NC, NS = 2, 16
NW = NC * NS
EPW = E // NW
CH = 125
BLK1 = 500
NBLK1 = EPW // BLK1
BLK2 = 250
NBLK2 = EPW // BLK2
ROWS_PT = N // NS

BE = 2000
BN = 2000

_f32 = jnp.float32


def _mesh():
    return plsc.VectorSubcoreMesh(
        core_axis_name="c", subcore_axis_name="s", num_cores=NC, num_subcores=NS)


def _proj_body(x_ref, wq_ref, wk_ref, wv_ref, q_ref, k_ref, v_ref):
    xv = x_ref[...]
    q_ref[...] = jnp.dot(xv, wq_ref[...], preferred_element_type=_f32)
    k_ref[...] = jnp.dot(xv, wk_ref[...], preferred_element_type=_f32)
    v_ref[...] = jnp.dot(xv, wv_ref[...], preferred_element_type=_f32)


def _tc_proj(x, WQ, WK, WV):
    out = jax.ShapeDtypeStruct((N, DI), _f32)
    return pl.pallas_call(
        _proj_body,
        out_shape=(out, out, out),
    )(x, WQ, WK, WV)


def _sc_gather_body(dst_hbm, src_hbm, qh_hbm, kh_hbm, msg_hbm,
                    dst_v, src_v, q_v, k_v, sem):
    c = lax.axis_index("c")
    s = lax.axis_index("s")
    wid = s * NC + c
    base = wid * EPW

    def blk(b, carry):
        off = base + b * BLK1
        roff = off // CH
        pltpu.sync_copy(dst_hbm.at[pl.ds(roff, BLK1 // CH)], dst_v)
        pltpu.sync_copy(src_hbm.at[pl.ds(roff, BLK1 // CH)], src_v)
        cps = []
        for j in range(BLK1 // CH):
            cps.append(pltpu.async_copy(
                qh_hbm.at[dst_v.at[j]], q_v.at[pl.ds(j * CH, CH)], sem))
            cps.append(pltpu.async_copy(
                kh_hbm.at[src_v.at[j]], k_v.at[pl.ds(j * CH, CH)], sem))
        for cp in cps:
            cp.wait()

        def row(i, rc):
            for j4 in range(DI // 16):
                sl = pl.ds(j4 * 16, 16)
                q_v[i, sl] = q_v[i, sl] + k_v[i, sl]
            return rc
        lax.fori_loop(0, BLK1, row, 0)
        pltpu.sync_copy(q_v, msg_hbm.at[pl.ds(off, BLK1)])
        return carry
    lax.fori_loop(0, NBLK1, blk, 0)


def _sc_gather(dst2d, src2d, Qh, Kh):
    kfn = pl.kernel(
        _sc_gather_body,
        out_type=jax.ShapeDtypeStruct((E, DI), _f32),
        mesh=_mesh(),
        scratch_types=[
            pltpu.VMEM((BLK1 // CH, CH), jnp.int32),
            pltpu.VMEM((BLK1 // CH, CH), jnp.int32),
            pltpu.VMEM((BLK1, DI), _f32),
            pltpu.VMEM((BLK1, DI), _f32),
            pltpu.SemaphoreType.DMA,
        ],
    )
    return kfn(dst2d, src2d, Qh, Kh)


def _edge_body(conn_ref, msg_ref, wew_ref, web_ref, beb_ref, weo_ref,
               beo_ref, awm_ref, e8_ref, oe_ref, p_ref):
    cb = conn_ref[...]
    ew = jnp.dot(cb, wew_ref[...], preferred_element_type=_f32)
    eb = jnp.dot(cb, web_ref[...], preferred_element_type=_f32) + beb_ref[...]
    m = msg_ref[...] * ew
    c2 = jnp.sign(m) * jnp.sqrt(jnp.abs(m))
    c3 = jnp.maximum(c2 + eb, 0.0)
    oe = jnp.dot(c3, weo_ref[...], preferred_element_type=_f32) + beo_ref[...]
    oe_ref[...] = oe
    sc = jnp.dot(oe, awm_ref[...], preferred_element_type=_f32)
    sc = jnp.clip(sc, -CLAMP, CLAMP)
    p8 = jnp.exp(sc)
    p_ref[...] = jnp.dot(p8, e8_ref[...], preferred_element_type=_f32)


def _tc_edge(conn, msg, WEw, WEb, bEb, WEo, bEo, Awm, E8):
    out_e = jax.ShapeDtypeStruct((E, DI), _f32)
    full = lambda shape: pl.BlockSpec(shape, lambda i: (0, 0))
    return pl.pallas_call(
        _edge_body,
        grid=(E // BE,),
        in_specs=[
            pl.BlockSpec((BE, HID), lambda i: (i, 0)),
            pl.BlockSpec((BE, DI), lambda i: (i, 0)),
            full((HID, DI)),
            full((HID, DI)),
            full((1, DI)),
            full((DI, DI)),
            full((1, DI)),
            full((DI, HEADS)),
            full((HEADS, DI)),
        ],
        out_specs=(pl.BlockSpec((BE, DI), lambda i: (i, 0)),
                   pl.BlockSpec((BE, DI), lambda i: (i, 0))),
        out_shape=(out_e, out_e),
    )(conn, msg, WEw, WEb, bEb, WEo, bEo, Awm, E8)


def _sc_scatter_body(dst_hbm, src_hbm, p_hbm, oe_hbm, vh_hbm, zz_hbm,
                     av_hbm, ac_hbm, ap_hbm,
                     dst_v, src_v, v_v, p_v, c_v, pv_v, pc_v, sem,
                     accv_sh, accc_sh, accp_sh):
    c = lax.axis_index("c")
    s = lax.axis_index("s")

    @pl.when(s == 0)
    def _():
        pltpu.sync_copy(zz_hbm, accv_sh)
        pltpu.sync_copy(zz_hbm, accc_sh)
        pltpu.sync_copy(zz_hbm, accp_sh)
    plsc.subcore_barrier()

    wid = s * NC + c
    base = wid * EPW

    def blk(b, carry):
        off = base + b * BLK2
        roff = off // CH
        pltpu.sync_copy(dst_hbm.at[pl.ds(roff, BLK2 // CH)], dst_v)
        pltpu.sync_copy(src_hbm.at[pl.ds(roff, BLK2 // CH)], src_v)
        cps = []
        for j in range(BLK2 // CH):
            cps.append(pltpu.async_copy(
                vh_hbm.at[src_v.at[j]], v_v.at[pl.ds(j * CH, CH)], sem))
        pltpu.sync_copy(p_hbm.at[pl.ds(off, BLK2)], p_v)
        pltpu.sync_copy(oe_hbm.at[pl.ds(off, BLK2)], c_v)
        for cp in cps:
            cp.wait()

        def row(i, rc):
            for j4 in range(DI // 16):
                sl = pl.ds(j4 * 16, 16)
                pv = p_v[i, sl]
                pv_v[i, sl] = pv * v_v[i, sl]
                pc_v[i, sl] = pv * c_v[i, sl]
            return rc
        lax.fori_loop(0, BLK2, row, 0)
        for j in range(BLK2 // CH):
            sl = pl.ds(j * CH, CH)
            pltpu.sync_copy(pv_v.at[sl], accv_sh.at[dst_v.at[j]], add=True)
            pltpu.sync_copy(pc_v.at[sl], accc_sh.at[dst_v.at[j]], add=True)
            pltpu.sync_copy(p_v.at[sl], accp_sh.at[dst_v.at[j]], add=True)
        return carry
    lax.fori_loop(0, NBLK2, blk, 0)
    plsc.subcore_barrier()

    r0 = s * ROWS_PT
    pltpu.sync_copy(accv_sh.at[pl.ds(r0, ROWS_PT)], av_hbm.at[c, pl.ds(r0, ROWS_PT)])
    pltpu.sync_copy(accc_sh.at[pl.ds(r0, ROWS_PT)], ac_hbm.at[c, pl.ds(r0, ROWS_PT)])
    pltpu.sync_copy(accp_sh.at[pl.ds(r0, ROWS_PT)], ap_hbm.at[c, pl.ds(r0, ROWS_PT)])


def _sc_scatter(dst2d, src2d, pexp, Oe, Vh, zz):
    out_acc = jax.ShapeDtypeStruct((NC, N, DI), _f32)
    kfn = pl.kernel(
        _sc_scatter_body,
        out_type=(out_acc, out_acc, out_acc),
        mesh=_mesh(),
        scratch_types=[
            pltpu.VMEM((BLK2 // CH, CH), jnp.int32),
            pltpu.VMEM((BLK2 // CH, CH), jnp.int32),
            pltpu.VMEM((BLK2, DI), _f32),
            pltpu.VMEM((BLK2, DI), _f32),
            pltpu.VMEM((BLK2, DI), _f32),
            pltpu.VMEM((BLK2, DI), _f32),
            pltpu.VMEM((BLK2, DI), _f32),
            pltpu.SemaphoreType.DMA,
            pltpu.VMEM_SHARED((N, DI), _f32),
            pltpu.VMEM_SHARED((N, DI), _f32),
            pltpu.VMEM_SHARED((N, DI), _f32),
        ],
    )
    return kfn(dst2d, src2d, pexp, Oe, Vh, zz)


def _comb_body(av_ref, ac_ref, ap_ref, bwm_ref, out_ref):
    ssum = ap_ref[0] + ap_ref[1] + 1e-16
    aggv = (av_ref[0] + av_ref[1]) / ssum
    aggc = (ac_ref[0] + ac_ref[1]) / ssum
    out_ref[...] = aggv + jnp.dot(aggc, bwm_ref[...], preferred_element_type=_f32)


def _tc_comb(av, ac, ap, BWm):
    spec_acc = pl.BlockSpec((NC, BN, DI), lambda i: (0, i, 0))
    return pl.pallas_call(
        _comb_body,
        grid=(N // BN,),
        in_specs=[spec_acc, spec_acc, spec_acc,
                  pl.BlockSpec((DI, DI), lambda i: (0, 0))],
        out_specs=pl.BlockSpec((BN, DI), lambda i: (i, 0)),
        out_shape=jax.ShapeDtypeStruct((N, DI), _f32),
    )(av, ac, ap, BWm)


def kernel(x, rrwp_index, rrwp_conn, WQ, WK, WV, WEw, WEb, bEb, WEo, bEo, Aw, BW):
    dst2d = rrwp_index[0].astype(jnp.int32).reshape(E // CH, CH)
    src2d = rrwp_index[1].astype(jnp.int32).reshape(E // CH, CH)

    eye = jnp.eye(HEADS, dtype=_f32)
    Awm = (Aw[:, :, 0].T[:, :, None] * eye[:, None, :]).reshape(DI, HEADS)
    E8 = jnp.kron(eye, jnp.ones((1, DIM), _f32))
    BWm = jnp.einsum('dhc,hg->hdgc', BW, eye).reshape(DI, DI)

    Qh, Kh, Vh = _tc_proj(x, WQ, WK, WV)
    msg = _sc_gather(dst2d, src2d, Qh, Kh)
    Oe, pexp = _tc_edge(rrwp_conn, msg, WEw, WEb, bEb.reshape(1, DI),
                        WEo, bEo.reshape(1, DI), Awm, E8)
    zz = jnp.zeros((N, DI), _f32)
    av, ac, ap = _sc_scatter(dst2d, src2d, pexp, Oe, Vh, zz)
    h_out = _tc_comb(av, ac, ap, BWm)
    return (h_out, Oe)


# speedup vs baseline: 56.0727x; 56.0727x over previous
import functools

import jax
import jax.numpy as jnp
from jax import lax
from jax.experimental import pallas as pl
from jax.experimental.pallas import tpu as pltpu
from jax.experimental.pallas import tpu_sc as plsc

N = 10000
E = 320000
HID = 128
HEADS = 8
DIM = 8
DI = HEADS * DIM
CLAMP = 5.0

NC, NS = 2, 16
NW = NC * NS
EPW = E // NW
CH = 125
BLK1 = 500
NBLK1 = EPW // BLK1
BLK2 = 250
NBLK2 = EPW // BLK2
ROWS_PT = N // NS

BE = 2000
BN = 2000

_f32 = jnp.float32


def _mesh():
    return plsc.VectorSubcoreMesh(
        core_axis_name="c", subcore_axis_name="s", num_cores=NC, num_subcores=NS)


def _proj_body(x_ref, wq_ref, wk_ref, wv_ref, q_ref, k_ref, v_ref):
    xv = x_ref[...]
    q_ref[...] = jnp.dot(xv, wq_ref[...], preferred_element_type=_f32)
    k_ref[...] = jnp.dot(xv, wk_ref[...], preferred_element_type=_f32)
    v_ref[...] = jnp.dot(xv, wv_ref[...], preferred_element_type=_f32)


def _tc_proj(x, WQ, WK, WV):
    out = jax.ShapeDtypeStruct((N, DI), _f32)
    return pl.pallas_call(
        _proj_body,
        out_shape=(out, out, out),
    )(x, WQ, WK, WV)


def _sc_gather_body(dst_hbm, src_hbm, qh_hbm, kh_hbm, msg_hbm,
                    dst_v, src_v, q_v, k_v, sem):
    c = lax.axis_index("c")
    s = lax.axis_index("s")
    wid = s * NC + c
    base = wid * EPW

    def blk(b, carry):
        off = base + b * BLK1
        roff = off // CH
        pltpu.sync_copy(dst_hbm.at[pl.ds(roff, BLK1 // CH)], dst_v)
        pltpu.sync_copy(src_hbm.at[pl.ds(roff, BLK1 // CH)], src_v)
        cps = []
        for j in range(BLK1 // CH):
            cps.append(pltpu.async_copy(
                qh_hbm.at[dst_v.at[j]], q_v.at[pl.ds(j * CH, CH)], sem))
            cps.append(pltpu.async_copy(
                kh_hbm.at[src_v.at[j]], k_v.at[pl.ds(j * CH, CH)], sem))
        for cp in cps:
            cp.wait()

        def row(i, rc):
            for j4 in range(DI // 16):
                sl = pl.ds(j4 * 16, 16)
                q_v[i, sl] = q_v[i, sl] + k_v[i, sl]
            return rc
        lax.fori_loop(0, BLK1, row, 0)
        pltpu.sync_copy(q_v, msg_hbm.at[pl.ds(off, BLK1)])
        return carry
    lax.fori_loop(0, NBLK1, blk, 0)


def _sc_gather(dst2d, src2d, Qh, Kh):
    kfn = pl.kernel(
        _sc_gather_body,
        out_type=jax.ShapeDtypeStruct((E, DI), _f32),
        mesh=_mesh(),
        scratch_types=[
            pltpu.VMEM((BLK1 // CH, CH), jnp.int32),
            pltpu.VMEM((BLK1 // CH, CH), jnp.int32),
            pltpu.VMEM((BLK1, DI), _f32),
            pltpu.VMEM((BLK1, DI), _f32),
            pltpu.SemaphoreType.DMA,
        ],
        compiler_params=pltpu.CompilerParams(use_tc_tiling_on_sc=False),
    )
    return kfn(dst2d, src2d, Qh, Kh)


def _edge_body(conn_ref, msg_ref, wew_ref, web_ref, beb_ref, weo_ref,
               beo_ref, awm_ref, e8_ref, i16_ref, oe_ref, p_ref, p16_ref, pc_ref):
    cb = conn_ref[...]
    ew = jnp.dot(cb, wew_ref[...], preferred_element_type=_f32)
    eb = jnp.dot(cb, web_ref[...], preferred_element_type=_f32) + beb_ref[...]
    m = msg_ref[...] * ew
    c2 = jnp.sign(m) * jnp.sqrt(jnp.abs(m))
    c3 = jnp.maximum(c2 + eb, 0.0)
    oe = jnp.dot(c3, weo_ref[...], preferred_element_type=_f32) + beo_ref[...]
    oe_ref[...] = oe
    sc = jnp.dot(oe, awm_ref[...], preferred_element_type=_f32)
    sc = jnp.clip(sc, -CLAMP, CLAMP)
    p8 = jnp.exp(sc)
    pexp = jnp.dot(p8, e8_ref[...], preferred_element_type=_f32)
    p_ref[...] = pexp
    p16_ref[...] = jnp.dot(p8, i16_ref[...], preferred_element_type=_f32)
    pc_ref[...] = pexp * oe


def _tc_edge(conn, msg, WEw, WEb, bEb, WEo, bEo, Awm, E8, I16):
    out_e = jax.ShapeDtypeStruct((E, DI), _f32)
    out_p16 = jax.ShapeDtypeStruct((E, 16), _f32)
    full = lambda shape: pl.BlockSpec(shape, lambda i: (0, 0))
    return pl.pallas_call(
        _edge_body,
        grid=(E // BE,),
        in_specs=[
            pl.BlockSpec((BE, HID), lambda i: (i, 0)),
            pl.BlockSpec((BE, DI), lambda i: (i, 0)),
            full((HID, DI)),
            full((HID, DI)),
            full((1, DI)),
            full((DI, DI)),
            full((1, DI)),
            full((DI, HEADS)),
            full((HEADS, DI)),
            full((HEADS, 16)),
        ],
        out_specs=(pl.BlockSpec((BE, DI), lambda i: (i, 0)),
                   pl.BlockSpec((BE, DI), lambda i: (i, 0)),
                   pl.BlockSpec((BE, 16), lambda i: (i, 0)),
                   pl.BlockSpec((BE, DI), lambda i: (i, 0))),
        out_shape=(out_e, out_e, out_p16, out_e),
    )(conn, msg, WEw, WEb, bEb, WEo, bEo, Awm, E8, I16)


def _zero_acc64(buf_v, acc_sh, r0):
    def zrow(i, rc):
        for j4 in range(DI // 16):
            buf_v[i, pl.ds(j4 * 16, 16)] = jnp.zeros((16,), _f32)
        return rc
    lax.fori_loop(0, BLK2, zrow, 0)
    pltpu.sync_copy(buf_v, acc_sh.at[pl.ds(r0, BLK2)])
    pltpu.sync_copy(buf_v, acc_sh.at[pl.ds(r0 + BLK2, BLK2)])
    pltpu.sync_copy(buf_v.at[pl.ds(0, ROWS_PT - 2 * BLK2)],
                    acc_sh.at[pl.ds(r0 + 2 * BLK2, ROWS_PT - 2 * BLK2)])


def _sc_scatter_v_body(dst_hbm, src_hbm, p_hbm, p16_hbm, vh_hbm,
                       av_hbm, ap_hbm,
                       dst_v, src_v, v_v, p_v, pv_v, p16_v, sem,
                       accv_sh, accp_sh):
    c = lax.axis_index("c")
    s = lax.axis_index("s")
    r0 = s * ROWS_PT
    _zero_acc64(pv_v, accv_sh, r0)

    def zrow16(i, rc):
        p16_v[i, pl.ds(0, 16)] = jnp.zeros((16,), _f32)
        return rc
    lax.fori_loop(0, BLK2, zrow16, 0)
    pltpu.sync_copy(p16_v, accp_sh.at[pl.ds(r0, BLK2)])
    pltpu.sync_copy(p16_v, accp_sh.at[pl.ds(r0 + BLK2, BLK2)])
    pltpu.sync_copy(p16_v.at[pl.ds(0, ROWS_PT - 2 * BLK2)],
                    accp_sh.at[pl.ds(r0 + 2 * BLK2, ROWS_PT - 2 * BLK2)])
    plsc.subcore_barrier()

    base = (s * NC + c) * EPW

    def blk(b, carry):
        off = base + b * BLK2
        roff = off // CH
        pltpu.sync_copy(dst_hbm.at[pl.ds(roff, BLK2 // CH)], dst_v)
        pltpu.sync_copy(src_hbm.at[pl.ds(roff, BLK2 // CH)], src_v)
        cps = []
        for j in range(BLK2 // CH):
            cps.append(pltpu.async_copy(
                vh_hbm.at[src_v.at[j]], v_v.at[pl.ds(j * CH, CH)], sem))
        pltpu.sync_copy(p_hbm.at[pl.ds(off, BLK2)], p_v)
        pltpu.sync_copy(p16_hbm.at[pl.ds(off, BLK2)], p16_v)
        for cp in cps:
            cp.wait()

        def row(i, rc):
            for j4 in range(DI // 16):
                sl = pl.ds(j4 * 16, 16)
                pv_v[i, sl] = p_v[i, sl] * v_v[i, sl]
            return rc
        lax.fori_loop(0, BLK2, row, 0)
        for j in range(BLK2 // CH):
            sl = pl.ds(j * CH, CH)
            pltpu.sync_copy(pv_v.at[sl], accv_sh.at[dst_v.at[j]], add=True)
            pltpu.sync_copy(p16_v.at[sl], accp_sh.at[dst_v.at[j]], add=True)
        return carry
    lax.fori_loop(0, NBLK2, blk, 0)
    plsc.subcore_barrier()

    pltpu.sync_copy(accv_sh.at[pl.ds(r0, ROWS_PT)], av_hbm.at[c, pl.ds(r0, ROWS_PT)])
    pltpu.sync_copy(accp_sh.at[pl.ds(r0, ROWS_PT)], ap_hbm.at[c, pl.ds(r0, ROWS_PT)])


def _sc_scatter_v(dst2d, src2d, pexp, p16, Vh):
    kfn = pl.kernel(
        _sc_scatter_v_body,
        out_type=(jax.ShapeDtypeStruct((NC, N, DI), _f32),
                  jax.ShapeDtypeStruct((NC, N, 16), _f32)),
        mesh=_mesh(),
        scratch_types=[
            pltpu.VMEM((BLK2 // CH, CH), jnp.int32),
            pltpu.VMEM((BLK2 // CH, CH), jnp.int32),
            pltpu.VMEM((BLK2, DI), _f32),
            pltpu.VMEM((BLK2, DI), _f32),
            pltpu.VMEM((BLK2, DI), _f32),
            pltpu.VMEM((BLK2, 16), _f32),
            pltpu.SemaphoreType.DMA,
            pltpu.VMEM_SHARED((N, DI), _f32),
            pltpu.VMEM_SHARED((N, 16), _f32),
        ],
        compiler_params=pltpu.CompilerParams(use_tc_tiling_on_sc=False),
    )
    return kfn(dst2d, src2d, pexp, p16, Vh)


def _sc_scatter_c_body(dst_hbm, pc_hbm, ac_hbm,
                       dst_v, pc_v, accc_sh):
    c = lax.axis_index("c")
    s = lax.axis_index("s")
    r0 = s * ROWS_PT
    _zero_acc64(pc_v, accc_sh, r0)
    plsc.subcore_barrier()

    base = (s * NC + c) * EPW

    def blk(b, carry):
        off = base + b * BLK2
        roff = off // CH
        pltpu.sync_copy(dst_hbm.at[pl.ds(roff, BLK2 // CH)], dst_v)
        pltpu.sync_copy(pc_hbm.at[pl.ds(off, BLK2)], pc_v)
        for j in range(BLK2 // CH):
            pltpu.sync_copy(pc_v.at[pl.ds(j * CH, CH)],
                            accc_sh.at[dst_v.at[j]], add=True)
        return carry
    lax.fori_loop(0, NBLK2, blk, 0)
    plsc.subcore_barrier()
    pltpu.sync_copy(accc_sh.at[pl.ds(r0, ROWS_PT)], ac_hbm.at[c, pl.ds(r0, ROWS_PT)])


def _sc_scatter_c(dst2d, pc):
    kfn = pl.kernel(
        _sc_scatter_c_body,
        out_type=jax.ShapeDtypeStruct((NC, N, DI), _f32),
        mesh=_mesh(),
        scratch_types=[
            pltpu.VMEM((BLK2 // CH, CH), jnp.int32),
            pltpu.VMEM((BLK2, DI), _f32),
            pltpu.VMEM_SHARED((N, DI), _f32),
        ],
        compiler_params=pltpu.CompilerParams(use_tc_tiling_on_sc=False),
    )
    return kfn(dst2d, pc)


def _comb_body(av_ref, ac_ref, ap_ref, bwm_ref, e16_ref, out_ref):
    w16 = 1.0 / (ap_ref[0] + ap_ref[1] + 1e-16)
    wexp = jnp.dot(w16, e16_ref[...], preferred_element_type=_f32)
    aggv = (av_ref[0] + av_ref[1]) * wexp
    aggc = (ac_ref[0] + ac_ref[1]) * wexp
    out_ref[...] = aggv + jnp.dot(aggc, bwm_ref[...], preferred_element_type=_f32)


def _tc_comb(av, ac, ap, BWm, E16):
    spec_acc = pl.BlockSpec((NC, BN, DI), lambda i: (0, i, 0))
    return pl.pallas_call(
        _comb_body,
        grid=(N // BN,),
        in_specs=[spec_acc, spec_acc,
                  pl.BlockSpec((NC, BN, 16), lambda i: (0, i, 0)),
                  pl.BlockSpec((DI, DI), lambda i: (0, 0)),
                  pl.BlockSpec((16, DI), lambda i: (0, 0))],
        out_specs=pl.BlockSpec((BN, DI), lambda i: (i, 0)),
        out_shape=jax.ShapeDtypeStruct((N, DI), _f32),
    )(av, ac, ap, BWm, E16)


def kernel(x, rrwp_index, rrwp_conn, WQ, WK, WV, WEw, WEb, bEb, WEo, bEo, Aw, BW):
    dst2d = rrwp_index[0].astype(jnp.int32).reshape(E // CH, CH)
    src2d = rrwp_index[1].astype(jnp.int32).reshape(E // CH, CH)

    eye = jnp.eye(HEADS, dtype=_f32)
    Awm = (Aw[:, :, 0].T[:, :, None] * eye[:, None, :]).reshape(DI, HEADS)
    E8 = jnp.kron(eye, jnp.ones((1, DIM), _f32))
    E16 = jnp.concatenate([E8, jnp.zeros((8, DI), _f32)], axis=0)
    I16 = jnp.concatenate([eye, jnp.zeros((8, 8), _f32)], axis=1)
    BWm = jnp.einsum('dhc,hg->hdgc', BW, eye).reshape(DI, DI)

    Qh, Kh, Vh = _tc_proj(x, WQ, WK, WV)
    msg = _sc_gather(dst2d, src2d, Qh, Kh)
    Oe, pexp, p16, pc = _tc_edge(rrwp_conn, msg, WEw, WEb, bEb.reshape(1, DI),
                                 WEo, bEo.reshape(1, DI), Awm, E8, I16)
    av, ap = _sc_scatter_v(dst2d, src2d, pexp, p16, Vh)
    ac = _sc_scatter_c(dst2d, pc)
    h_out = _tc_comb(av, ac, ap, BWm, E16)
    return (h_out, Oe)


# speedup vs baseline: 64.2130x; 1.1452x over previous
import functools

import jax
import jax.numpy as jnp
from jax import lax
from jax.experimental import pallas as pl
from jax.experimental.pallas import tpu as pltpu
from jax.experimental.pallas import tpu_sc as plsc

N = 10000
E = 320000
HID = 128
HEADS = 8
DIM = 8
DI = HEADS * DIM
CLAMP = 5.0

NC, NS = 2, 16
NW = NC * NS
EPW = E // NW
CH = 125
BLK1 = 500
NBLK1 = EPW // BLK1
BLK2 = 250
NBLK2 = EPW // BLK2
ROWS_PT = N // NS

BE = 2000
BN = 2000

_f32 = jnp.float32


def _mesh():
    return plsc.VectorSubcoreMesh(
        core_axis_name="c", subcore_axis_name="s", num_cores=NC, num_subcores=NS)


def _proj_body(x_ref, wq_ref, wk_ref, wv_ref, q_ref, k_ref, v_ref):
    xv = x_ref[...]
    q_ref[...] = jnp.dot(xv, wq_ref[...], preferred_element_type=_f32)
    k_ref[...] = jnp.dot(xv, wk_ref[...], preferred_element_type=_f32)
    v_ref[...] = jnp.dot(xv, wv_ref[...], preferred_element_type=_f32)


def _tc_proj(x, WQ, WK, WV):
    out = jax.ShapeDtypeStruct((N, DI), _f32)
    return pl.pallas_call(
        _proj_body,
        out_shape=(out, out, out),
    )(x, WQ, WK, WV)


def _sc_gather_body(dst_hbm, src_hbm, qh_hbm, kh_hbm, msg_hbm,
                    dst_v, src_v, q_v, k_v, sem):
    c = lax.axis_index("c")
    s = lax.axis_index("s")
    wid = s * NC + c
    base = wid * EPW

    def blk(b, carry):
        off = base + b * BLK1
        roff = off // CH
        pltpu.sync_copy(dst_hbm.at[pl.ds(roff, BLK1 // CH)], dst_v)
        pltpu.sync_copy(src_hbm.at[pl.ds(roff, BLK1 // CH)], src_v)
        cps = []
        for j in range(BLK1 // CH):
            cps.append(pltpu.async_copy(
                qh_hbm.at[dst_v.at[j]], q_v.at[pl.ds(j * CH, CH)], sem))
            cps.append(pltpu.async_copy(
                kh_hbm.at[src_v.at[j]], k_v.at[pl.ds(j * CH, CH)], sem))
        for cp in cps:
            cp.wait()

        def row(i, rc):
            for j4 in range(DI // 16):
                sl = pl.ds(j4 * 16, 16)
                q_v[i, sl] = q_v[i, sl] + k_v[i, sl]
            return rc
        lax.fori_loop(0, BLK1, row, 0)
        pltpu.sync_copy(q_v, msg_hbm.at[pl.ds(off, BLK1)])
        return carry
    lax.fori_loop(0, NBLK1, blk, 0)


def _sc_gather(dst2d, src2d, Qh, Kh):
    kfn = pl.kernel(
        _sc_gather_body,
        out_type=jax.ShapeDtypeStruct((E, DI), _f32),
        mesh=_mesh(),
        scratch_types=[
            pltpu.VMEM((BLK1 // CH, CH), jnp.int32),
            pltpu.VMEM((BLK1 // CH, CH), jnp.int32),
            pltpu.VMEM((BLK1, DI), _f32),
            pltpu.VMEM((BLK1, DI), _f32),
            pltpu.SemaphoreType.DMA,
        ],
        compiler_params=pltpu.CompilerParams(use_tc_tiling_on_sc=False),
    )
    return kfn(dst2d, src2d, Qh, Kh)


def _edge_body(conn_ref, msg_ref, wew_ref, web_ref, beb_ref, weo_ref,
               beo_ref, awm_ref, e8_ref, i16_ref, oe_ref, p16_ref, pc_ref):
    cb = conn_ref[...]
    ew = jnp.dot(cb, wew_ref[...], preferred_element_type=_f32)
    eb = jnp.dot(cb, web_ref[...], preferred_element_type=_f32) + beb_ref[...]
    m = msg_ref[...] * ew
    c2 = jnp.sign(m) * jnp.sqrt(jnp.abs(m))
    c3 = jnp.maximum(c2 + eb, 0.0)
    oe = jnp.dot(c3, weo_ref[...], preferred_element_type=_f32) + beo_ref[...]
    oe_ref[...] = oe
    sc = jnp.dot(oe, awm_ref[...], preferred_element_type=_f32)
    sc = jnp.clip(sc, -CLAMP, CLAMP)
    p8 = jnp.exp(sc)
    pexp = jnp.dot(p8, e8_ref[...], preferred_element_type=_f32)
    p16_ref[...] = jnp.dot(p8, i16_ref[...], preferred_element_type=_f32)
    pc_ref[...] = pexp * oe


def _tc_edge(conn, msg, WEw, WEb, bEb, WEo, bEo, Awm, E8, I16):
    out_e = jax.ShapeDtypeStruct((E, DI), _f32)
    out_p16 = jax.ShapeDtypeStruct((E, 16), _f32)
    full = lambda shape: pl.BlockSpec(shape, lambda i: (0, 0))
    return pl.pallas_call(
        _edge_body,
        grid=(E // BE,),
        in_specs=[
            pl.BlockSpec((BE, HID), lambda i: (i, 0)),
            pl.BlockSpec((BE, DI), lambda i: (i, 0)),
            full((HID, DI)),
            full((HID, DI)),
            full((1, DI)),
            full((DI, DI)),
            full((1, DI)),
            full((DI, HEADS)),
            full((HEADS, DI)),
            full((HEADS, 16)),
        ],
        out_specs=(pl.BlockSpec((BE, DI), lambda i: (i, 0)),
                   pl.BlockSpec((BE, 16), lambda i: (i, 0)),
                   pl.BlockSpec((BE, DI), lambda i: (i, 0))),
        out_shape=(out_e, out_p16, out_e),
    )(conn, msg, WEw, WEb, bEb, WEo, bEo, Awm, E8, I16)


def _zero_acc64(buf_v, acc_sh, r0):
    def zrow(i, rc):
        for j4 in range(DI // 16):
            buf_v[i, pl.ds(j4 * 16, 16)] = jnp.zeros((16,), _f32)
        return rc
    lax.fori_loop(0, BLK2, zrow, 0)
    pltpu.sync_copy(buf_v, acc_sh.at[pl.ds(r0, BLK2)])
    pltpu.sync_copy(buf_v, acc_sh.at[pl.ds(r0 + BLK2, BLK2)])
    pltpu.sync_copy(buf_v.at[pl.ds(0, ROWS_PT - 2 * BLK2)],
                    acc_sh.at[pl.ds(r0 + 2 * BLK2, ROWS_PT - 2 * BLK2)])


def _sc_scatter_v_body(dst_hbm, src_hbm, p16_hbm, vh_hbm,
                       av_hbm, ap_hbm,
                       dst_v, src_v, v_v, pv_v, p16_v, sem,
                       accv_sh, accp_sh):
    c = lax.axis_index("c")
    s = lax.axis_index("s")
    r0 = s * ROWS_PT
    _zero_acc64(pv_v, accv_sh, r0)

    def zrow16(i, rc):
        p16_v[i, pl.ds(0, 16)] = jnp.zeros((16,), _f32)
        return rc
    lax.fori_loop(0, BLK2, zrow16, 0)
    pltpu.sync_copy(p16_v, accp_sh.at[pl.ds(r0, BLK2)])
    pltpu.sync_copy(p16_v, accp_sh.at[pl.ds(r0 + BLK2, BLK2)])
    pltpu.sync_copy(p16_v.at[pl.ds(0, ROWS_PT - 2 * BLK2)],
                    accp_sh.at[pl.ds(r0 + 2 * BLK2, ROWS_PT - 2 * BLK2)])
    plsc.subcore_barrier()

    base = (s * NC + c) * EPW
    hofb = lax.iota(jnp.int32, 16) >= DIM

    def blk(b, carry):
        off = base + b * BLK2
        roff = off // CH
        pltpu.sync_copy(dst_hbm.at[pl.ds(roff, BLK2 // CH)], dst_v)
        pltpu.sync_copy(src_hbm.at[pl.ds(roff, BLK2 // CH)], src_v)
        cps = []
        for j in range(BLK2 // CH):
            cps.append(pltpu.async_copy(
                vh_hbm.at[src_v.at[j]], v_v.at[pl.ds(j * CH, CH)], sem))
        pltpu.sync_copy(p16_hbm.at[pl.ds(off, BLK2)], p16_v)
        for cp in cps:
            cp.wait()

        def row(i, rc):
            pr = p16_v[i, pl.ds(0, 16)]
            for j4 in range(DI // 16):
                sl = pl.ds(j4 * 16, 16)
                pb = jnp.where(hofb,
                               jnp.full((16,), pr[2 * j4 + 1], _f32),
                               jnp.full((16,), pr[2 * j4], _f32))
                pv_v[i, sl] = pb * v_v[i, sl]
            return rc
        lax.fori_loop(0, BLK2, row, 0)
        for j in range(BLK2 // CH):
            sl = pl.ds(j * CH, CH)
            pltpu.sync_copy(pv_v.at[sl], accv_sh.at[dst_v.at[j]], add=True)
            pltpu.sync_copy(p16_v.at[sl], accp_sh.at[dst_v.at[j]], add=True)
        return carry
    lax.fori_loop(0, NBLK2, blk, 0)
    plsc.subcore_barrier()

    pltpu.sync_copy(accv_sh.at[pl.ds(r0, ROWS_PT)], av_hbm.at[c, pl.ds(r0, ROWS_PT)])
    pltpu.sync_copy(accp_sh.at[pl.ds(r0, ROWS_PT)], ap_hbm.at[c, pl.ds(r0, ROWS_PT)])


def _sc_scatter_v(dst2d, src2d, p16, Vh):
    kfn = pl.kernel(
        _sc_scatter_v_body,
        out_type=(jax.ShapeDtypeStruct((NC, N, DI), _f32),
                  jax.ShapeDtypeStruct((NC, N, 16), _f32)),
        mesh=_mesh(),
        scratch_types=[
            pltpu.VMEM((BLK2 // CH, CH), jnp.int32),
            pltpu.VMEM((BLK2 // CH, CH), jnp.int32),
            pltpu.VMEM((BLK2, DI), _f32),
            pltpu.VMEM((BLK2, DI), _f32),
            pltpu.VMEM((BLK2, 16), _f32),
            pltpu.SemaphoreType.DMA,
            pltpu.VMEM_SHARED((N, DI), _f32),
            pltpu.VMEM_SHARED((N, 16), _f32),
        ],
        compiler_params=pltpu.CompilerParams(use_tc_tiling_on_sc=False),
    )
    return kfn(dst2d, src2d, p16, Vh)


def _sc_scatter_c_body(dst_hbm, pc_hbm, ac_hbm,
                       dst_v, pc_v, accc_sh):
    c = lax.axis_index("c")
    s = lax.axis_index("s")
    r0 = s * ROWS_PT
    _zero_acc64(pc_v, accc_sh, r0)
    plsc.subcore_barrier()

    base = (s * NC + c) * EPW

    def blk(b, carry):
        off = base + b * BLK2
        roff = off // CH
        pltpu.sync_copy(dst_hbm.at[pl.ds(roff, BLK2 // CH)], dst_v)
        pltpu.sync_copy(pc_hbm.at[pl.ds(off, BLK2)], pc_v)
        for j in range(BLK2 // CH):
            pltpu.sync_copy(pc_v.at[pl.ds(j * CH, CH)],
                            accc_sh.at[dst_v.at[j]], add=True)
        return carry
    lax.fori_loop(0, NBLK2, blk, 0)
    plsc.subcore_barrier()
    pltpu.sync_copy(accc_sh.at[pl.ds(r0, ROWS_PT)], ac_hbm.at[c, pl.ds(r0, ROWS_PT)])


def _sc_scatter_c(dst2d, pc):
    kfn = pl.kernel(
        _sc_scatter_c_body,
        out_type=jax.ShapeDtypeStruct((NC, N, DI), _f32),
        mesh=_mesh(),
        scratch_types=[
            pltpu.VMEM((BLK2 // CH, CH), jnp.int32),
            pltpu.VMEM((BLK2, DI), _f32),
            pltpu.VMEM_SHARED((N, DI), _f32),
        ],
        compiler_params=pltpu.CompilerParams(use_tc_tiling_on_sc=False),
    )
    return kfn(dst2d, pc)


def _comb_body(av_ref, ac_ref, ap_ref, bwm_ref, e16_ref, out_ref):
    w16 = 1.0 / (ap_ref[0] + ap_ref[1] + 1e-16)
    wexp = jnp.dot(w16, e16_ref[...], preferred_element_type=_f32)
    aggv = (av_ref[0] + av_ref[1]) * wexp
    aggc = (ac_ref[0] + ac_ref[1]) * wexp
    out_ref[...] = aggv + jnp.dot(aggc, bwm_ref[...], preferred_element_type=_f32)


def _tc_comb(av, ac, ap, BWm, E16):
    spec_acc = pl.BlockSpec((NC, BN, DI), lambda i: (0, i, 0))
    return pl.pallas_call(
        _comb_body,
        grid=(N // BN,),
        in_specs=[spec_acc, spec_acc,
                  pl.BlockSpec((NC, BN, 16), lambda i: (0, i, 0)),
                  pl.BlockSpec((DI, DI), lambda i: (0, 0)),
                  pl.BlockSpec((16, DI), lambda i: (0, 0))],
        out_specs=pl.BlockSpec((BN, DI), lambda i: (i, 0)),
        out_shape=jax.ShapeDtypeStruct((N, DI), _f32),
    )(av, ac, ap, BWm, E16)


def kernel(x, rrwp_index, rrwp_conn, WQ, WK, WV, WEw, WEb, bEb, WEo, bEo, Aw, BW):
    dst2d = rrwp_index[0].astype(jnp.int32).reshape(E // CH, CH)
    src2d = rrwp_index[1].astype(jnp.int32).reshape(E // CH, CH)

    eye = jnp.eye(HEADS, dtype=_f32)
    Awm = (Aw[:, :, 0].T[:, :, None] * eye[:, None, :]).reshape(DI, HEADS)
    E8 = jnp.kron(eye, jnp.ones((1, DIM), _f32))
    E16 = jnp.concatenate([E8, jnp.zeros((8, DI), _f32)], axis=0)
    I16 = jnp.concatenate([eye, jnp.zeros((8, 8), _f32)], axis=1)
    BWm = jnp.einsum('dhc,hg->hdgc', BW, eye).reshape(DI, DI)

    Qh, Kh, Vh = _tc_proj(x, WQ, WK, WV)
    msg = _sc_gather(dst2d, src2d, Qh, Kh)
    Oe, p16, pc = _tc_edge(rrwp_conn, msg, WEw, WEb, bEb.reshape(1, DI),
                           WEo, bEo.reshape(1, DI), Awm, E8, I16)
    av, ap = _sc_scatter_v(dst2d, src2d, p16, Vh)
    ac = _sc_scatter_c(dst2d, pc)
    h_out = _tc_comb(av, ac, ap, BWm, E16)
    return (h_out, Oe)


# speedup vs baseline: 70.8182x; 1.1029x over previous
import functools

import jax
import jax.numpy as jnp
from jax import lax
from jax.experimental import pallas as pl
from jax.experimental.pallas import tpu as pltpu
from jax.experimental.pallas import tpu_sc as plsc

N = 10000
E = 320000
HID = 128
HEADS = 8
DIM = 8
DI = HEADS * DIM
CLAMP = 5.0

NC, NS = 2, 16
NW = NC * NS
EPW = E // NW
CH = 125
BLK1 = 500
NBLK1 = EPW // BLK1
BLK2 = 250
NBLK2 = EPW // BLK2
BV = 125
NBV = EPW // BV
GV = NBV // 4
ROWS_PT = N // NS

BE = 2000
BN = 2000

_f32 = jnp.float32


def _mesh():
    return plsc.VectorSubcoreMesh(
        core_axis_name="c", subcore_axis_name="s", num_cores=NC, num_subcores=NS)


def _proj_body(x_ref, wq_ref, wk_ref, wv_ref, q_ref, k_ref, v_ref):
    xv = x_ref[...]
    q_ref[...] = jnp.dot(xv, wq_ref[...], preferred_element_type=_f32)
    k_ref[...] = jnp.dot(xv, wk_ref[...], preferred_element_type=_f32)
    v_ref[...] = jnp.dot(xv, wv_ref[...], preferred_element_type=_f32)


def _tc_proj(x, WQ, WK, WV):
    out = jax.ShapeDtypeStruct((N, DI), _f32)
    return pl.pallas_call(
        _proj_body,
        out_shape=(out, out, out),
    )(x, WQ, WK, WV)


def _sc_gather_body(dst_hbm, src_hbm, qh_hbm, kh_hbm, msg_hbm,
                    dst_v, src_v, q_v, k_v, sem):
    c = lax.axis_index("c")
    s = lax.axis_index("s")
    wid = s * NC + c
    base = wid * EPW

    def blk(b, carry):
        off = base + b * BLK1
        roff = off // CH
        pltpu.sync_copy(dst_hbm.at[pl.ds(roff, BLK1 // CH)], dst_v)
        pltpu.sync_copy(src_hbm.at[pl.ds(roff, BLK1 // CH)], src_v)
        cps = []
        for j in range(BLK1 // CH):
            cps.append(pltpu.async_copy(
                qh_hbm.at[dst_v.at[j]], q_v.at[pl.ds(j * CH, CH)], sem))
            cps.append(pltpu.async_copy(
                kh_hbm.at[src_v.at[j]], k_v.at[pl.ds(j * CH, CH)], sem))
        for cp in cps:
            cp.wait()

        def row(i, rc):
            for j4 in range(DI // 16):
                sl = pl.ds(j4 * 16, 16)
                q_v[i, sl] = q_v[i, sl] + k_v[i, sl]
            return rc
        lax.fori_loop(0, BLK1, row, 0)
        pltpu.sync_copy(q_v, msg_hbm.at[pl.ds(off, BLK1)])
        return carry
    lax.fori_loop(0, NBLK1, blk, 0)


def _sc_gather(dst2d, src2d, Qh, Kh):
    kfn = pl.kernel(
        _sc_gather_body,
        out_type=jax.ShapeDtypeStruct((E, DI), _f32),
        mesh=_mesh(),
        scratch_types=[
            pltpu.VMEM((BLK1 // CH, CH), jnp.int32),
            pltpu.VMEM((BLK1 // CH, CH), jnp.int32),
            pltpu.VMEM((BLK1, DI), _f32),
            pltpu.VMEM((BLK1, DI), _f32),
            pltpu.SemaphoreType.DMA,
        ],
        compiler_params=pltpu.CompilerParams(use_tc_tiling_on_sc=False),
    )
    return kfn(dst2d, src2d, Qh, Kh)


def _edge_body(conn_ref, msg_ref, wew_ref, web_ref, beb_ref, weo_ref,
               beo_ref, awm_ref, e8_ref, i16_ref, oe_ref, p16_ref, pc_ref):
    cb = conn_ref[...]
    ew = jnp.dot(cb, wew_ref[...], preferred_element_type=_f32)
    eb = jnp.dot(cb, web_ref[...], preferred_element_type=_f32) + beb_ref[...]
    m = msg_ref[...] * ew
    c2 = jnp.sign(m) * jnp.sqrt(jnp.abs(m))
    c3 = jnp.maximum(c2 + eb, 0.0)
    oe = jnp.dot(c3, weo_ref[...], preferred_element_type=_f32) + beo_ref[...]
    oe_ref[...] = oe
    sc = jnp.dot(oe, awm_ref[...], preferred_element_type=_f32)
    sc = jnp.clip(sc, -CLAMP, CLAMP)
    p8 = jnp.exp(sc)
    pexp = jnp.dot(p8, e8_ref[...], preferred_element_type=_f32)
    p16_ref[...] = jnp.dot(p8, i16_ref[...], preferred_element_type=_f32)
    pc_ref[...] = pexp * oe


def _tc_edge(conn, msg, WEw, WEb, bEb, WEo, bEo, Awm, E8, I16):
    out_e = jax.ShapeDtypeStruct((E, DI), _f32)
    out_p16 = jax.ShapeDtypeStruct((E, 16), _f32)
    full = lambda shape: pl.BlockSpec(shape, lambda i: (0, 0))
    return pl.pallas_call(
        _edge_body,
        grid=(E // BE,),
        in_specs=[
            pl.BlockSpec((BE, HID), lambda i: (i, 0)),
            pl.BlockSpec((BE, DI), lambda i: (i, 0)),
            full((HID, DI)),
            full((HID, DI)),
            full((1, DI)),
            full((DI, DI)),
            full((1, DI)),
            full((DI, HEADS)),
            full((HEADS, DI)),
            full((HEADS, 16)),
        ],
        out_specs=(pl.BlockSpec((BE, DI), lambda i: (i, 0)),
                   pl.BlockSpec((BE, 16), lambda i: (i, 0)),
                   pl.BlockSpec((BE, DI), lambda i: (i, 0))),
        out_shape=(out_e, out_p16, out_e),
    )(conn, msg, WEw, WEb, bEb, WEo, bEo, Awm, E8, I16)


def _zero_acc64(buf_v, acc_sh, r0):
    def zrow(i, rc):
        for j4 in range(DI // 16):
            buf_v[i, pl.ds(j4 * 16, 16)] = jnp.zeros((16,), _f32)
        return rc
    lax.fori_loop(0, BLK2, zrow, 0)
    pltpu.sync_copy(buf_v, acc_sh.at[pl.ds(r0, BLK2)])
    pltpu.sync_copy(buf_v, acc_sh.at[pl.ds(r0 + BLK2, BLK2)])
    pltpu.sync_copy(buf_v.at[pl.ds(0, ROWS_PT - 2 * BLK2)],
                    acc_sh.at[pl.ds(r0 + 2 * BLK2, ROWS_PT - 2 * BLK2)])


def _sc_scatter_v_body(dst_hbm, src_hbm, p16_hbm, vh_hbm,
                       av_hbm, ap_hbm,
                       dst_v, src_v, v_v, pv_v, p16_v,
                       isem0, isem1, isem2, isem3,
                       dsem0, dsem1, dsem2, dsem3,
                       ssem0, ssem1, ssem2, ssem3,
                       accv_sh, accp_sh):
    c = lax.axis_index("c")
    s = lax.axis_index("s")
    isems = (isem0, isem1, isem2, isem3)
    dsems = (dsem0, dsem1, dsem2, dsem3)
    ssems = (ssem0, ssem1, ssem2, ssem3)
    r0 = s * ROWS_PT

    def zrow(i, rc):
        for j4 in range(DI // 16):
            pv_v[0, i, pl.ds(j4 * 16, 16)] = jnp.zeros((16,), _f32)
        p16_v[0, i, pl.ds(0, 16)] = jnp.zeros((16,), _f32)
        return rc
    lax.fori_loop(0, BV, zrow, 0)
    for t in range(ROWS_PT // BV):
        pltpu.sync_copy(pv_v.at[0], accv_sh.at[pl.ds(r0 + t * BV, BV)])
        pltpu.sync_copy(p16_v.at[0], accp_sh.at[pl.ds(r0 + t * BV, BV)])
    plsc.subcore_barrier()

    base = (s * NC + c) * EPW
    hofb = lax.iota(jnp.int32, 16) >= DIM

    def idx_cps(blk_i, k):
        roff = base // CH + blk_i
        return [pltpu.make_async_copy(dst_hbm.at[pl.ds(roff, 1)],
                                      dst_v.at[pl.ds(k, 1)], isems[k]),
                pltpu.make_async_copy(src_hbm.at[pl.ds(roff, 1)],
                                      src_v.at[pl.ds(k, 1)], isems[k])]

    def data_cps(blk_i, k):
        off = base + blk_i * BV
        return [pltpu.make_async_copy(vh_hbm.at[src_v.at[k]], v_v.at[k], dsems[k]),
                pltpu.make_async_copy(p16_hbm.at[pl.ds(off, BV)],
                                      p16_v.at[k], dsems[k])]

    def sc_cps(k):
        return [pltpu.make_async_copy(pv_v.at[k], accv_sh.at[dst_v.at[k]], ssems[k]),
                pltpu.make_async_copy(p16_v.at[k], accp_sh.at[dst_v.at[k]], ssems[k])]

    def sc_issue(k):
        pltpu.async_copy(pv_v.at[k], accv_sh.at[dst_v.at[k]], ssems[k], add=True)
        pltpu.async_copy(p16_v.at[k], accp_sh.at[dst_v.at[k]], ssems[k], add=True)

    for d in idx_cps(0, 0):
        d.start()
    for d in idx_cps(1, 1):
        d.start()
    for d in idx_cps(0, 0):
        d.wait()
    for d in data_cps(0, 0):
        d.start()

    def gbody(g, carry):
        for k in range(4):
            cur = g * 4 + k
            k2 = (k + 2) % 4
            k1 = (k + 1) % 4
            if k < 2:
                @pl.when(g > 0)
                def _(k2=k2):
                    for d in sc_cps(k2):
                        d.wait()
            else:
                for d in sc_cps(k2):
                    d.wait()
            if k < 2:
                for d in idx_cps(cur + 2, k2):
                    d.start()
            else:
                @pl.when(g < GV - 1)
                def _(cur=cur, k2=k2):
                    for d in idx_cps(cur + 2, k2):
                        d.start()
            if k < 3:
                for d in idx_cps(cur + 1, k1):
                    d.wait()
                for d in data_cps(cur + 1, k1):
                    d.start()
            else:
                @pl.when(g < GV - 1)
                def _(cur=cur, k1=k1):
                    for d in idx_cps(cur + 1, k1):
                        d.wait()
                    for d in data_cps(cur + 1, k1):
                        d.start()
            for d in data_cps(cur, k):
                d.wait()

            def row(i, rc, k=k):
                pr = p16_v[k, i, pl.ds(0, 16)]
                for j4 in range(DI // 16):
                    sl = pl.ds(j4 * 16, 16)
                    pb = jnp.where(hofb,
                                   jnp.full((16,), pr[2 * j4 + 1], _f32),
                                   jnp.full((16,), pr[2 * j4], _f32))
                    pv_v[k, i, sl] = pb * v_v[k, i, sl]
                return rc
            lax.fori_loop(0, BV, row, 0)
            sc_issue(k)
        return carry
    lax.fori_loop(0, GV, gbody, 0)
    for k in (2, 3):
        for d in sc_cps(k):
            d.wait()
    plsc.subcore_barrier()

    pltpu.sync_copy(accv_sh.at[pl.ds(r0, ROWS_PT)], av_hbm.at[c, pl.ds(r0, ROWS_PT)])
    pltpu.sync_copy(accp_sh.at[pl.ds(r0, ROWS_PT)], ap_hbm.at[c, pl.ds(r0, ROWS_PT)])


def _sc_scatter_v(dst2d, src2d, p16, Vh):
    kfn = pl.kernel(
        _sc_scatter_v_body,
        out_type=(jax.ShapeDtypeStruct((NC, N, DI), _f32),
                  jax.ShapeDtypeStruct((NC, N, 16), _f32)),
        mesh=_mesh(),
        scratch_types=[
            pltpu.VMEM((4, CH), jnp.int32),
            pltpu.VMEM((4, CH), jnp.int32),
            pltpu.VMEM((4, BV, DI), _f32),
            pltpu.VMEM((4, BV, DI), _f32),
            pltpu.VMEM((4, BV, 16), _f32),
        ] + [pltpu.SemaphoreType.DMA] * 12 + [
            pltpu.VMEM_SHARED((N, DI), _f32),
            pltpu.VMEM_SHARED((N, 16), _f32),
        ],
        compiler_params=pltpu.CompilerParams(use_tc_tiling_on_sc=False),
    )
    return kfn(dst2d, src2d, p16, Vh)


def _sc_scatter_c_body(dst_hbm, pc_hbm, ac_hbm,
                       dst_v, pc_v, accc_sh):
    c = lax.axis_index("c")
    s = lax.axis_index("s")
    r0 = s * ROWS_PT
    _zero_acc64(pc_v, accc_sh, r0)
    plsc.subcore_barrier()

    base = (s * NC + c) * EPW

    def blk(b, carry):
        off = base + b * BLK2
        roff = off // CH
        pltpu.sync_copy(dst_hbm.at[pl.ds(roff, BLK2 // CH)], dst_v)
        pltpu.sync_copy(pc_hbm.at[pl.ds(off, BLK2)], pc_v)
        for j in range(BLK2 // CH):
            pltpu.sync_copy(pc_v.at[pl.ds(j * CH, CH)],
                            accc_sh.at[dst_v.at[j]], add=True)
        return carry
    lax.fori_loop(0, NBLK2, blk, 0)
    plsc.subcore_barrier()
    pltpu.sync_copy(accc_sh.at[pl.ds(r0, ROWS_PT)], ac_hbm.at[c, pl.ds(r0, ROWS_PT)])


def _sc_scatter_c(dst2d, pc):
    kfn = pl.kernel(
        _sc_scatter_c_body,
        out_type=jax.ShapeDtypeStruct((NC, N, DI), _f32),
        mesh=_mesh(),
        scratch_types=[
            pltpu.VMEM((BLK2 // CH, CH), jnp.int32),
            pltpu.VMEM((BLK2, DI), _f32),
            pltpu.VMEM_SHARED((N, DI), _f32),
        ],
        compiler_params=pltpu.CompilerParams(use_tc_tiling_on_sc=False),
    )
    return kfn(dst2d, pc)


def _comb_body(av_ref, ac_ref, ap_ref, bwm_ref, e16_ref, out_ref):
    w16 = 1.0 / (ap_ref[0] + ap_ref[1] + 1e-16)
    wexp = jnp.dot(w16, e16_ref[...], preferred_element_type=_f32)
    aggv = (av_ref[0] + av_ref[1]) * wexp
    aggc = (ac_ref[0] + ac_ref[1]) * wexp
    out_ref[...] = aggv + jnp.dot(aggc, bwm_ref[...], preferred_element_type=_f32)


def _tc_comb(av, ac, ap, BWm, E16):
    spec_acc = pl.BlockSpec((NC, BN, DI), lambda i: (0, i, 0))
    return pl.pallas_call(
        _comb_body,
        grid=(N // BN,),
        in_specs=[spec_acc, spec_acc,
                  pl.BlockSpec((NC, BN, 16), lambda i: (0, i, 0)),
                  pl.BlockSpec((DI, DI), lambda i: (0, 0)),
                  pl.BlockSpec((16, DI), lambda i: (0, 0))],
        out_specs=pl.BlockSpec((BN, DI), lambda i: (i, 0)),
        out_shape=jax.ShapeDtypeStruct((N, DI), _f32),
    )(av, ac, ap, BWm, E16)


def kernel(x, rrwp_index, rrwp_conn, WQ, WK, WV, WEw, WEb, bEb, WEo, bEo, Aw, BW):
    dst2d = rrwp_index[0].astype(jnp.int32).reshape(E // CH, CH)
    src2d = rrwp_index[1].astype(jnp.int32).reshape(E // CH, CH)

    eye = jnp.eye(HEADS, dtype=_f32)
    Awm = (Aw[:, :, 0].T[:, :, None] * eye[:, None, :]).reshape(DI, HEADS)
    E8 = jnp.kron(eye, jnp.ones((1, DIM), _f32))
    E16 = jnp.concatenate([E8, jnp.zeros((8, DI), _f32)], axis=0)
    I16 = jnp.concatenate([eye, jnp.zeros((8, 8), _f32)], axis=1)
    BWm = jnp.einsum('dhc,hg->hdgc', BW, eye).reshape(DI, DI)

    Qh, Kh, Vh = _tc_proj(x, WQ, WK, WV)
    msg = _sc_gather(dst2d, src2d, Qh, Kh)
    Oe, p16, pc = _tc_edge(rrwp_conn, msg, WEw, WEb, bEb.reshape(1, DI),
                           WEo, bEo.reshape(1, DI), Awm, E8, I16)
    av, ap = _sc_scatter_v(dst2d, src2d, p16, Vh)
    ac = _sc_scatter_c(dst2d, pc)
    h_out = _tc_comb(av, ac, ap, BWm, E16)
    return (h_out, Oe)


# speedup vs baseline: 75.1035x; 1.0605x over previous
import functools

import jax
import jax.numpy as jnp
from jax import lax
from jax.experimental import pallas as pl
from jax.experimental.pallas import tpu as pltpu
from jax.experimental.pallas import tpu_sc as plsc

N = 10000
E = 320000
HID = 128
HEADS = 8
DIM = 8
DI = HEADS * DIM
CLAMP = 5.0

NC, NS = 2, 16
NW = NC * NS
EPW = E // NW
CH = 125
BV = 125
NBV = EPW // BV
GV = NBV // 4
BG = 125
GG = (EPW // BG) // 4
ROWS_PT = N // NS

BE = 2000
BN = 2000

_f32 = jnp.float32


def _mesh():
    return plsc.VectorSubcoreMesh(
        core_axis_name="c", subcore_axis_name="s", num_cores=NC, num_subcores=NS)


def _proj_body(x_ref, wq_ref, wk_ref, wv_ref, q_ref, k_ref, v_ref):
    xv = x_ref[...]
    q_ref[...] = jnp.dot(xv, wq_ref[...], preferred_element_type=_f32)
    k_ref[...] = jnp.dot(xv, wk_ref[...], preferred_element_type=_f32)
    v_ref[...] = jnp.dot(xv, wv_ref[...], preferred_element_type=_f32)


def _tc_proj(x, WQ, WK, WV):
    out = jax.ShapeDtypeStruct((N, DI), _f32)
    return pl.pallas_call(
        _proj_body,
        out_shape=(out, out, out),
    )(x, WQ, WK, WV)


def _sc_gather_body(dst_hbm, src_hbm, qh_hbm, kh_hbm, msg_hbm,
                    dst_v, src_v, q_v, k_v,
                    isem0, isem1, isem2, isem3,
                    dsem0, dsem1, dsem2, dsem3,
                    osem0, osem1, osem2, osem3):
    c = lax.axis_index("c")
    s = lax.axis_index("s")
    isems = (isem0, isem1, isem2, isem3)
    dsems = (dsem0, dsem1, dsem2, dsem3)
    osems = (osem0, osem1, osem2, osem3)
    base = (s * NC + c) * EPW

    def idx_cps(blk_i, k):
        roff = base // CH + blk_i
        return [pltpu.make_async_copy(dst_hbm.at[pl.ds(roff, 1)],
                                      dst_v.at[pl.ds(k, 1)], isems[k]),
                pltpu.make_async_copy(src_hbm.at[pl.ds(roff, 1)],
                                      src_v.at[pl.ds(k, 1)], isems[k])]

    def data_cps(blk_i, k):
        return [pltpu.make_async_copy(qh_hbm.at[dst_v.at[k]], q_v.at[k], dsems[k]),
                pltpu.make_async_copy(kh_hbm.at[src_v.at[k]], k_v.at[k], dsems[k])]

    def out_cp(blk_i, k):
        off = base + blk_i * BG
        return pltpu.make_async_copy(q_v.at[k], msg_hbm.at[pl.ds(off, BG)], osems[k])

    for d in idx_cps(0, 0):
        d.start()
    for d in idx_cps(1, 1):
        d.start()
    for d in idx_cps(0, 0):
        d.wait()
    for d in data_cps(0, 0):
        d.start()

    def gbody(g, carry):
        for k in range(4):
            cur = g * 4 + k
            k1 = (k + 1) % 4
            k2 = (k + 2) % 4
            if k < 2:
                for d in idx_cps(cur + 2, k2):
                    d.start()
            else:
                @pl.when(g < GG - 1)
                def _(cur=cur, k2=k2):
                    for d in idx_cps(cur + 2, k2):
                        d.start()
            if k < 3:
                for d in idx_cps(cur + 1, k1):
                    d.wait()

                @pl.when(g > 0)
                def _(cur=cur, k1=k1):
                    out_cp(cur - 3, k1).wait()
                for d in data_cps(cur + 1, k1):
                    d.start()
            else:
                @pl.when(g < GG - 1)
                def _(cur=cur, k1=k1):
                    for d in idx_cps(cur + 1, k1):
                        d.wait()
                    out_cp(cur - 3, k1).wait()
                    for d in data_cps(cur + 1, k1):
                        d.start()
            for d in data_cps(cur, k):
                d.wait()

            def row(i, rc, k=k):
                for j4 in range(DI // 16):
                    sl = pl.ds(j4 * 16, 16)
                    q_v[k, i, sl] = q_v[k, i, sl] + k_v[k, i, sl]
                return rc
            lax.fori_loop(0, BG, row, 0)
            out_cp(cur, k).start()
        return carry
    lax.fori_loop(0, GG, gbody, 0)
    for k in range(4):
        out_cp(4 * (GG - 1) + k, k).wait()


def _sc_gather(dst2d, src2d, Qh, Kh):
    kfn = pl.kernel(
        _sc_gather_body,
        out_type=jax.ShapeDtypeStruct((E, DI), _f32),
        mesh=_mesh(),
        scratch_types=[
            pltpu.VMEM((4, CH), jnp.int32),
            pltpu.VMEM((4, CH), jnp.int32),
            pltpu.VMEM((4, BG, DI), _f32),
            pltpu.VMEM((4, BG, DI), _f32),
        ] + [pltpu.SemaphoreType.DMA] * 12,
        compiler_params=pltpu.CompilerParams(use_tc_tiling_on_sc=False),
    )
    return kfn(dst2d, src2d, Qh, Kh)


def _edge_body(conn_ref, msg_ref, wew_ref, web_ref, beb_ref, weo_ref,
               beo_ref, awm_ref, e8_ref, i16_ref, oe_ref, p16_ref, pc_ref):
    cb = conn_ref[...]
    ew = jnp.dot(cb, wew_ref[...], preferred_element_type=_f32)
    eb = jnp.dot(cb, web_ref[...], preferred_element_type=_f32) + beb_ref[...]
    m = msg_ref[...] * ew
    c2 = jnp.sign(m) * jnp.sqrt(jnp.abs(m))
    c3 = jnp.maximum(c2 + eb, 0.0)
    oe = jnp.dot(c3, weo_ref[...], preferred_element_type=_f32) + beo_ref[...]
    oe_ref[...] = oe
    sc = jnp.dot(oe, awm_ref[...], preferred_element_type=_f32)
    sc = jnp.clip(sc, -CLAMP, CLAMP)
    p8 = jnp.exp(sc)
    pexp = jnp.dot(p8, e8_ref[...], preferred_element_type=_f32)
    p16_ref[...] = jnp.dot(p8, i16_ref[...], preferred_element_type=_f32)
    pc_ref[...] = pexp * oe


def _tc_edge(conn, msg, WEw, WEb, bEb, WEo, bEo, Awm, E8, I16):
    out_e = jax.ShapeDtypeStruct((E, DI), _f32)
    out_p16 = jax.ShapeDtypeStruct((E, 16), _f32)
    full = lambda shape: pl.BlockSpec(shape, lambda i: (0, 0))
    return pl.pallas_call(
        _edge_body,
        grid=(E // BE,),
        in_specs=[
            pl.BlockSpec((BE, HID), lambda i: (i, 0)),
            pl.BlockSpec((BE, DI), lambda i: (i, 0)),
            full((HID, DI)),
            full((HID, DI)),
            full((1, DI)),
            full((DI, DI)),
            full((1, DI)),
            full((DI, HEADS)),
            full((HEADS, DI)),
            full((HEADS, 16)),
        ],
        out_specs=(pl.BlockSpec((BE, DI), lambda i: (i, 0)),
                   pl.BlockSpec((BE, 16), lambda i: (i, 0)),
                   pl.BlockSpec((BE, DI), lambda i: (i, 0))),
        out_shape=(out_e, out_p16, out_e),
    )(conn, msg, WEw, WEb, bEb, WEo, bEo, Awm, E8, I16)


def _sc_scatter_v_body(dst_hbm, src_hbm, p16_hbm, vh_hbm,
                       av_hbm, ap_hbm,
                       dst_v, src_v, v_v, pv_v, p16_v,
                       isem0, isem1, isem2, isem3,
                       dsem0, dsem1, dsem2, dsem3,
                       ssem0, ssem1, ssem2, ssem3,
                       accv_sh, accp_sh):
    c = lax.axis_index("c")
    s = lax.axis_index("s")
    isems = (isem0, isem1, isem2, isem3)
    dsems = (dsem0, dsem1, dsem2, dsem3)
    ssems = (ssem0, ssem1, ssem2, ssem3)
    r0 = s * ROWS_PT

    def zrow(i, rc):
        for j4 in range(DI // 16):
            pv_v[0, i, pl.ds(j4 * 16, 16)] = jnp.zeros((16,), _f32)
        p16_v[0, i, pl.ds(0, 16)] = jnp.zeros((16,), _f32)
        return rc
    lax.fori_loop(0, BV, zrow, 0)
    for t in range(ROWS_PT // BV):
        pltpu.sync_copy(pv_v.at[0], accv_sh.at[pl.ds(r0 + t * BV, BV)])
        pltpu.sync_copy(p16_v.at[0], accp_sh.at[pl.ds(r0 + t * BV, BV)])
    plsc.subcore_barrier()

    base = (s * NC + c) * EPW
    hofb = lax.iota(jnp.int32, 16) >= DIM

    def idx_cps(blk_i, k):
        roff = base // CH + blk_i
        return [pltpu.make_async_copy(dst_hbm.at[pl.ds(roff, 1)],
                                      dst_v.at[pl.ds(k, 1)], isems[k]),
                pltpu.make_async_copy(src_hbm.at[pl.ds(roff, 1)],
                                      src_v.at[pl.ds(k, 1)], isems[k])]

    def data_cps(blk_i, k):
        off = base + blk_i * BV
        return [pltpu.make_async_copy(vh_hbm.at[src_v.at[k]], v_v.at[k], dsems[k]),
                pltpu.make_async_copy(p16_hbm.at[pl.ds(off, BV)],
                                      p16_v.at[k], dsems[k])]

    def sc_cps(k):
        return [pltpu.make_async_copy(pv_v.at[k], accv_sh.at[dst_v.at[k]], ssems[k]),
                pltpu.make_async_copy(p16_v.at[k], accp_sh.at[dst_v.at[k]], ssems[k])]

    def sc_issue(k):
        pltpu.async_copy(pv_v.at[k], accv_sh.at[dst_v.at[k]], ssems[k], add=True)
        pltpu.async_copy(p16_v.at[k], accp_sh.at[dst_v.at[k]], ssems[k], add=True)

    for d in idx_cps(0, 0):
        d.start()
    for d in idx_cps(1, 1):
        d.start()
    for d in idx_cps(0, 0):
        d.wait()
    for d in data_cps(0, 0):
        d.start()

    def gbody(g, carry):
        for k in range(4):
            cur = g * 4 + k
            k2 = (k + 2) % 4
            k1 = (k + 1) % 4
            if k < 2:
                @pl.when(g > 0)
                def _(k2=k2):
                    for d in sc_cps(k2):
                        d.wait()
            else:
                for d in sc_cps(k2):
                    d.wait()
            if k < 2:
                for d in idx_cps(cur + 2, k2):
                    d.start()
            else:
                @pl.when(g < GV - 1)
                def _(cur=cur, k2=k2):
                    for d in idx_cps(cur + 2, k2):
                        d.start()
            if k < 3:
                for d in idx_cps(cur + 1, k1):
                    d.wait()
                for d in data_cps(cur + 1, k1):
                    d.start()
            else:
                @pl.when(g < GV - 1)
                def _(cur=cur, k1=k1):
                    for d in idx_cps(cur + 1, k1):
                        d.wait()
                    for d in data_cps(cur + 1, k1):
                        d.start()
            for d in data_cps(cur, k):
                d.wait()

            def row(i, rc, k=k):
                pr = p16_v[k, i, pl.ds(0, 16)]
                for j4 in range(DI // 16):
                    sl = pl.ds(j4 * 16, 16)
                    pb = jnp.where(hofb,
                                   jnp.full((16,), pr[2 * j4 + 1], _f32),
                                   jnp.full((16,), pr[2 * j4], _f32))
                    pv_v[k, i, sl] = pb * v_v[k, i, sl]
                return rc
            lax.fori_loop(0, BV, row, 0)
            sc_issue(k)
        return carry
    lax.fori_loop(0, GV, gbody, 0)
    for k in (2, 3):
        for d in sc_cps(k):
            d.wait()
    plsc.subcore_barrier()

    pltpu.sync_copy(accv_sh.at[pl.ds(r0, ROWS_PT)], av_hbm.at[c, pl.ds(r0, ROWS_PT)])
    pltpu.sync_copy(accp_sh.at[pl.ds(r0, ROWS_PT)], ap_hbm.at[c, pl.ds(r0, ROWS_PT)])


def _sc_scatter_v(dst2d, src2d, p16, Vh):
    kfn = pl.kernel(
        _sc_scatter_v_body,
        out_type=(jax.ShapeDtypeStruct((NC, N, DI), _f32),
                  jax.ShapeDtypeStruct((NC, N, 16), _f32)),
        mesh=_mesh(),
        scratch_types=[
            pltpu.VMEM((4, CH), jnp.int32),
            pltpu.VMEM((4, CH), jnp.int32),
            pltpu.VMEM((4, BV, DI), _f32),
            pltpu.VMEM((4, BV, DI), _f32),
            pltpu.VMEM((4, BV, 16), _f32),
        ] + [pltpu.SemaphoreType.DMA] * 12 + [
            pltpu.VMEM_SHARED((N, DI), _f32),
            pltpu.VMEM_SHARED((N, 16), _f32),
        ],
        compiler_params=pltpu.CompilerParams(use_tc_tiling_on_sc=False),
    )
    return kfn(dst2d, src2d, p16, Vh)


def _sc_scatter_c_body(dst_hbm, pc_hbm, ac_hbm,
                       dst_v, pc_v,
                       isem0, isem1, isem2, isem3,
                       dsem0, dsem1, dsem2, dsem3,
                       ssem0, ssem1, ssem2, ssem3,
                       accc_sh):
    c = lax.axis_index("c")
    s = lax.axis_index("s")
    isems = (isem0, isem1, isem2, isem3)
    dsems = (dsem0, dsem1, dsem2, dsem3)
    ssems = (ssem0, ssem1, ssem2, ssem3)
    r0 = s * ROWS_PT

    def zrow(i, rc):
        for j4 in range(DI // 16):
            pc_v[0, i, pl.ds(j4 * 16, 16)] = jnp.zeros((16,), _f32)
        return rc
    lax.fori_loop(0, BV, zrow, 0)
    for t in range(ROWS_PT // BV):
        pltpu.sync_copy(pc_v.at[0], accc_sh.at[pl.ds(r0 + t * BV, BV)])
    plsc.subcore_barrier()

    base = (s * NC + c) * EPW

    def idx_cps(blk_i, k):
        roff = base // CH + blk_i
        return [pltpu.make_async_copy(dst_hbm.at[pl.ds(roff, 1)],
                                      dst_v.at[pl.ds(k, 1)], isems[k])]

    def data_cps(blk_i, k):
        off = base + blk_i * BV
        return [pltpu.make_async_copy(pc_hbm.at[pl.ds(off, BV)],
                                      pc_v.at[k], dsems[k])]

    def sc_cps(k):
        return [pltpu.make_async_copy(pc_v.at[k], accc_sh.at[dst_v.at[k]], ssems[k])]

    for d in idx_cps(0, 0):
        d.start()
    for d in idx_cps(1, 1):
        d.start()
    for d in idx_cps(0, 0):
        d.wait()
    for d in data_cps(0, 0):
        d.start()

    def gbody(g, carry):
        for k in range(4):
            cur = g * 4 + k
            k1 = (k + 1) % 4
            k2 = (k + 2) % 4
            if k < 2:
                @pl.when(g > 0)
                def _(k2=k2):
                    for d in sc_cps(k2):
                        d.wait()
            else:
                for d in sc_cps(k2):
                    d.wait()
            if k < 2:
                for d in idx_cps(cur + 2, k2):
                    d.start()
            else:
                @pl.when(g < GV - 1)
                def _(cur=cur, k2=k2):
                    for d in idx_cps(cur + 2, k2):
                        d.start()
            if k < 3:
                for d in idx_cps(cur + 1, k1):
                    d.wait()
                for d in data_cps(cur + 1, k1):
                    d.start()
            else:
                @pl.when(g < GV - 1)
                def _(cur=cur, k1=k1):
                    for d in idx_cps(cur + 1, k1):
                        d.wait()
                    for d in data_cps(cur + 1, k1):
                        d.start()
            for d in data_cps(cur, k):
                d.wait()
            pltpu.async_copy(pc_v.at[k], accc_sh.at[dst_v.at[k]], ssems[k], add=True)
        return carry
    lax.fori_loop(0, GV, gbody, 0)
    for k in (2, 3):
        for d in sc_cps(k):
            d.wait()
    plsc.subcore_barrier()
    pltpu.sync_copy(accc_sh.at[pl.ds(r0, ROWS_PT)], ac_hbm.at[c, pl.ds(r0, ROWS_PT)])


def _sc_scatter_c(dst2d, pc):
    kfn = pl.kernel(
        _sc_scatter_c_body,
        out_type=jax.ShapeDtypeStruct((NC, N, DI), _f32),
        mesh=_mesh(),
        scratch_types=[
            pltpu.VMEM((4, CH), jnp.int32),
            pltpu.VMEM((4, BV, DI), _f32),
        ] + [pltpu.SemaphoreType.DMA] * 12 + [
            pltpu.VMEM_SHARED((N, DI), _f32),
        ],
        compiler_params=pltpu.CompilerParams(use_tc_tiling_on_sc=False),
    )
    return kfn(dst2d, pc)


def _comb_body(av_ref, ac_ref, ap_ref, bwm_ref, e16_ref, out_ref):
    w16 = 1.0 / (ap_ref[0] + ap_ref[1] + 1e-16)
    wexp = jnp.dot(w16, e16_ref[...], preferred_element_type=_f32)
    aggv = (av_ref[0] + av_ref[1]) * wexp
    aggc = (ac_ref[0] + ac_ref[1]) * wexp
    out_ref[...] = aggv + jnp.dot(aggc, bwm_ref[...], preferred_element_type=_f32)


def _tc_comb(av, ac, ap, BWm, E16):
    spec_acc = pl.BlockSpec((NC, BN, DI), lambda i: (0, i, 0))
    return pl.pallas_call(
        _comb_body,
        grid=(N // BN,),
        in_specs=[spec_acc, spec_acc,
                  pl.BlockSpec((NC, BN, 16), lambda i: (0, i, 0)),
                  pl.BlockSpec((DI, DI), lambda i: (0, 0)),
                  pl.BlockSpec((16, DI), lambda i: (0, 0))],
        out_specs=pl.BlockSpec((BN, DI), lambda i: (i, 0)),
        out_shape=jax.ShapeDtypeStruct((N, DI), _f32),
    )(av, ac, ap, BWm, E16)


def kernel(x, rrwp_index, rrwp_conn, WQ, WK, WV, WEw, WEb, bEb, WEo, bEo, Aw, BW):
    dst2d = rrwp_index[0].astype(jnp.int32).reshape(E // CH, CH)
    src2d = rrwp_index[1].astype(jnp.int32).reshape(E // CH, CH)

    eye = jnp.eye(HEADS, dtype=_f32)
    Awm = (Aw[:, :, 0].T[:, :, None] * eye[:, None, :]).reshape(DI, HEADS)
    E8 = jnp.kron(eye, jnp.ones((1, DIM), _f32))
    E16 = jnp.concatenate([E8, jnp.zeros((8, DI), _f32)], axis=0)
    I16 = jnp.concatenate([eye, jnp.zeros((8, 8), _f32)], axis=1)
    BWm = jnp.einsum('dhc,hg->hdgc', BW, eye).reshape(DI, DI)

    Qh, Kh, Vh = _tc_proj(x, WQ, WK, WV)
    msg = _sc_gather(dst2d, src2d, Qh, Kh)
    Oe, p16, pc = _tc_edge(rrwp_conn, msg, WEw, WEb, bEb.reshape(1, DI),
                           WEo, bEo.reshape(1, DI), Awm, E8, I16)
    av, ap = _sc_scatter_v(dst2d, src2d, p16, Vh)
    ac = _sc_scatter_c(dst2d, pc)
    h_out = _tc_comb(av, ac, ap, BWm, E16)
    return (h_out, Oe)


# speedup vs baseline: 122.8473x; 1.6357x over previous
import functools

import jax
import jax.numpy as jnp
from jax import lax
from jax.experimental import pallas as pl
from jax.experimental.pallas import tpu as pltpu
from jax.experimental.pallas import tpu_sc as plsc

N = 10000
E = 320000
HID = 128
HEADS = 8
DIM = 8
DI = HEADS * DIM
CLAMP = 5.0

NC, NS = 2, 16
NW = NC * NS
EPW = E // NW
CH = 125
BV = 125
NBV = EPW // BV
GV = NBV // 4
BG = 125
GG = (EPW // BG) // 4
ROWS_PT = N // NS

BE = 2560
BN = 2000

_f32 = jnp.float32


def _mesh():
    return plsc.VectorSubcoreMesh(
        core_axis_name="c", subcore_axis_name="s", num_cores=NC, num_subcores=NS)


def _proj_body(x_ref, wq_ref, wk_ref, wv_ref, q_ref, k_ref, v_ref):
    xv = x_ref[...]
    q_ref[...] = jnp.dot(xv, wq_ref[...], preferred_element_type=_f32)
    k_ref[...] = jnp.dot(xv, wk_ref[...], preferred_element_type=_f32)
    v_ref[...] = jnp.dot(xv, wv_ref[...], preferred_element_type=_f32)


def _tc_proj(x, WQ, WK, WV):
    out = jax.ShapeDtypeStruct((N, DI), _f32)
    return pl.pallas_call(
        _proj_body,
        out_shape=(out, out, out),
    )(x, WQ, WK, WV)


def _sc_gather_body(dst_hbm, src_hbm, qh_hbm, kh_hbm, msg_hbm,
                    dst_v, src_v, q_v, k_v,
                    isem0, isem1, isem2, isem3,
                    dsem0, dsem1, dsem2, dsem3,
                    osem0, osem1, osem2, osem3):
    c = lax.axis_index("c")
    s = lax.axis_index("s")
    isems = (isem0, isem1, isem2, isem3)
    dsems = (dsem0, dsem1, dsem2, dsem3)
    osems = (osem0, osem1, osem2, osem3)
    base = (s * NC + c) * EPW

    def idx_cps(blk_i, k):
        roff = base // CH + blk_i
        return [pltpu.make_async_copy(dst_hbm.at[pl.ds(roff, 1)],
                                      dst_v.at[pl.ds(k, 1)], isems[k]),
                pltpu.make_async_copy(src_hbm.at[pl.ds(roff, 1)],
                                      src_v.at[pl.ds(k, 1)], isems[k])]

    def data_cps(blk_i, k):
        return [pltpu.make_async_copy(qh_hbm.at[dst_v.at[k]], q_v.at[k], dsems[k]),
                pltpu.make_async_copy(kh_hbm.at[src_v.at[k]], k_v.at[k], dsems[k])]

    def out_cp(blk_i, k):
        off = base + blk_i * BG
        return pltpu.make_async_copy(
            q_v.at[k], msg_hbm.at[pl.ds(off, BG), pl.ds(0, DI)], osems[k])

    for d in idx_cps(0, 0):
        d.start()
    for d in idx_cps(1, 1):
        d.start()
    for d in idx_cps(0, 0):
        d.wait()
    for d in data_cps(0, 0):
        d.start()

    def gbody(g, carry):
        for k in range(4):
            cur = g * 4 + k
            k1 = (k + 1) % 4
            k2 = (k + 2) % 4
            if k < 2:
                for d in idx_cps(cur + 2, k2):
                    d.start()
            else:
                @pl.when(g < GG - 1)
                def _(cur=cur, k2=k2):
                    for d in idx_cps(cur + 2, k2):
                        d.start()
            if k < 3:
                for d in idx_cps(cur + 1, k1):
                    d.wait()

                @pl.when(g > 0)
                def _(cur=cur, k1=k1):
                    out_cp(cur - 3, k1).wait()
                for d in data_cps(cur + 1, k1):
                    d.start()
            else:
                @pl.when(g < GG - 1)
                def _(cur=cur, k1=k1):
                    for d in idx_cps(cur + 1, k1):
                        d.wait()
                    out_cp(cur - 3, k1).wait()
                    for d in data_cps(cur + 1, k1):
                        d.start()
            for d in data_cps(cur, k):
                d.wait()

            def row(i, rc, k=k):
                for j4 in range(DI // 16):
                    sl = pl.ds(j4 * 16, 16)
                    q_v[k, i, sl] = q_v[k, i, sl] + k_v[k, i, sl]
                return rc
            lax.fori_loop(0, BG, row, 0)
            out_cp(cur, k).start()
        return carry
    lax.fori_loop(0, GG, gbody, 0)
    for k in range(4):
        out_cp(4 * (GG - 1) + k, k).wait()


def _sc_gather(dst2d, src2d, Qh, Kh):
    kfn = pl.kernel(
        _sc_gather_body,
        out_type=jax.ShapeDtypeStruct((E, HID), _f32),
        mesh=_mesh(),
        scratch_types=[
            pltpu.VMEM((4, CH), jnp.int32),
            pltpu.VMEM((4, CH), jnp.int32),
            pltpu.VMEM((4, BG, DI), _f32),
            pltpu.VMEM((4, BG, DI), _f32),
        ] + [pltpu.SemaphoreType.DMA] * 12,
        compiler_params=pltpu.CompilerParams(use_tc_tiling_on_sc=False),
    )
    return kfn(dst2d, src2d, Qh, Kh)


def _edge_body(conn_ref, msg_ref, wew_ref, web_ref, beb_ref, weo_ref,
               beo_ref, awm_ref, e8_ref, i16_ref, oe_ref, pc_ref):
    cb = conn_ref[...]
    ew = jnp.dot(cb, wew_ref[...], preferred_element_type=_f32)
    eb = jnp.dot(cb, web_ref[...], preferred_element_type=_f32) + beb_ref[...]
    m = msg_ref[:, :DI] * ew
    c2 = jnp.sign(m) * jnp.sqrt(jnp.abs(m))
    c3 = jnp.maximum(c2 + eb, 0.0)
    oe = jnp.dot(c3, weo_ref[...], preferred_element_type=_f32) + beo_ref[...]
    oe_ref[...] = oe.T
    sc = jnp.dot(oe, awm_ref[...], preferred_element_type=_f32)
    sc = jnp.clip(sc, -CLAMP, CLAMP)
    p8 = jnp.exp(sc)
    pexp = jnp.dot(p8, e8_ref[...], preferred_element_type=_f32)
    p16v = jnp.dot(p8, i16_ref[...], preferred_element_type=_f32)
    pc_ref[...] = jnp.concatenate(
        [pexp * oe, p16v, jnp.zeros((BE, HID - DI - 16), _f32)], axis=1)


def _tc_edge(conn, msg, WEw, WEb, bEb, WEo, bEo, Awm, E8, I16):
    out_e = jax.ShapeDtypeStruct((DI, E), _f32)
    out_pc = jax.ShapeDtypeStruct((E, HID), _f32)
    full = lambda shape: pl.BlockSpec(shape, lambda i: (0, 0))
    return pl.pallas_call(
        _edge_body,
        grid=(E // BE,),
        in_specs=[
            pl.BlockSpec((BE, HID), lambda i: (i, 0)),
            pl.BlockSpec((BE, HID), lambda i: (i, 0)),
            full((HID, DI)),
            full((HID, DI)),
            full((1, DI)),
            full((DI, DI)),
            full((1, DI)),
            full((DI, HEADS)),
            full((HEADS, DI)),
            full((HEADS, 16)),
        ],
        out_specs=(pl.BlockSpec((DI, BE), lambda i: (0, i)),
                   pl.BlockSpec((BE, HID), lambda i: (i, 0))),
        out_shape=(out_e, out_pc),
    )(conn, msg, WEw, WEb, bEb, WEo, bEo, Awm, E8, I16)


def _sc_scatter_v_body(dst_hbm, src_hbm, pc_hbm, vh_hbm,
                       av_hbm, ap_hbm,
                       dst_v, src_v, v_v, pv_v, p16_v,
                       isem0, isem1, isem2, isem3,
                       dsem0, dsem1, dsem2, dsem3,
                       ssem0, ssem1, ssem2, ssem3,
                       accv_sh, accp_sh):
    c = lax.axis_index("c")
    s = lax.axis_index("s")
    isems = (isem0, isem1, isem2, isem3)
    dsems = (dsem0, dsem1, dsem2, dsem3)
    ssems = (ssem0, ssem1, ssem2, ssem3)
    r0 = s * ROWS_PT

    def zrow(i, rc):
        for j4 in range(DI // 16):
            pv_v[0, i, pl.ds(j4 * 16, 16)] = jnp.zeros((16,), _f32)
        p16_v[0, i, pl.ds(0, 16)] = jnp.zeros((16,), _f32)
        return rc
    lax.fori_loop(0, BV, zrow, 0)
    for t in range(ROWS_PT // BV):
        pltpu.sync_copy(pv_v.at[0], accv_sh.at[pl.ds(r0 + t * BV, BV)])
        pltpu.sync_copy(p16_v.at[0], accp_sh.at[pl.ds(r0 + t * BV, BV)])
    plsc.subcore_barrier()

    base = (s * NC + c) * EPW
    hofb = lax.iota(jnp.int32, 16) >= DIM

    def idx_cps(blk_i, k):
        roff = base // CH + blk_i
        return [pltpu.make_async_copy(dst_hbm.at[pl.ds(roff, 1)],
                                      dst_v.at[pl.ds(k, 1)], isems[k]),
                pltpu.make_async_copy(src_hbm.at[pl.ds(roff, 1)],
                                      src_v.at[pl.ds(k, 1)], isems[k])]

    def data_cps(blk_i, k):
        off = base + blk_i * BV
        return [pltpu.make_async_copy(vh_hbm.at[src_v.at[k]], v_v.at[k], dsems[k]),
                pltpu.make_async_copy(pc_hbm.at[pl.ds(off, BV), pl.ds(DI, 16)],
                                      p16_v.at[k], dsems[k])]

    def sc_cps(k):
        return [pltpu.make_async_copy(pv_v.at[k], accv_sh.at[dst_v.at[k]], ssems[k]),
                pltpu.make_async_copy(p16_v.at[k], accp_sh.at[dst_v.at[k]], ssems[k])]

    def sc_issue(k):
        pltpu.async_copy(pv_v.at[k], accv_sh.at[dst_v.at[k]], ssems[k], add=True)
        pltpu.async_copy(p16_v.at[k], accp_sh.at[dst_v.at[k]], ssems[k], add=True)

    for d in idx_cps(0, 0):
        d.start()
    for d in idx_cps(1, 1):
        d.start()
    for d in idx_cps(0, 0):
        d.wait()
    for d in data_cps(0, 0):
        d.start()

    def gbody(g, carry):
        for k in range(4):
            cur = g * 4 + k
            k2 = (k + 2) % 4
            k1 = (k + 1) % 4
            if k < 2:
                @pl.when(g > 0)
                def _(k2=k2):
                    for d in sc_cps(k2):
                        d.wait()
            else:
                for d in sc_cps(k2):
                    d.wait()
            if k < 2:
                for d in idx_cps(cur + 2, k2):
                    d.start()
            else:
                @pl.when(g < GV - 1)
                def _(cur=cur, k2=k2):
                    for d in idx_cps(cur + 2, k2):
                        d.start()
            if k < 3:
                for d in idx_cps(cur + 1, k1):
                    d.wait()
                for d in data_cps(cur + 1, k1):
                    d.start()
            else:
                @pl.when(g < GV - 1)
                def _(cur=cur, k1=k1):
                    for d in idx_cps(cur + 1, k1):
                        d.wait()
                    for d in data_cps(cur + 1, k1):
                        d.start()
            for d in data_cps(cur, k):
                d.wait()

            def row(i, rc, k=k):
                pr = p16_v[k, i, pl.ds(0, 16)]
                for j4 in range(DI // 16):
                    sl = pl.ds(j4 * 16, 16)
                    pb = jnp.where(hofb,
                                   jnp.full((16,), pr[2 * j4 + 1], _f32),
                                   jnp.full((16,), pr[2 * j4], _f32))
                    pv_v[k, i, sl] = pb * v_v[k, i, sl]
                return rc
            lax.fori_loop(0, BV, row, 0)
            sc_issue(k)
        return carry
    lax.fori_loop(0, GV, gbody, 0)
    for k in (2, 3):
        for d in sc_cps(k):
            d.wait()
    plsc.subcore_barrier()

    pltpu.sync_copy(accv_sh.at[pl.ds(r0, ROWS_PT)], av_hbm.at[c, pl.ds(r0, ROWS_PT)])
    pltpu.sync_copy(accp_sh.at[pl.ds(r0, ROWS_PT)], ap_hbm.at[c, pl.ds(r0, ROWS_PT)])


def _sc_scatter_v(dst2d, src2d, pcfull, Vh):
    kfn = pl.kernel(
        _sc_scatter_v_body,
        out_type=(jax.ShapeDtypeStruct((NC, N, DI), _f32),
                  jax.ShapeDtypeStruct((NC, N, 16), _f32)),
        mesh=_mesh(),
        scratch_types=[
            pltpu.VMEM((4, CH), jnp.int32),
            pltpu.VMEM((4, CH), jnp.int32),
            pltpu.VMEM((4, BV, DI), _f32),
            pltpu.VMEM((4, BV, DI), _f32),
            pltpu.VMEM((4, BV, 16), _f32),
        ] + [pltpu.SemaphoreType.DMA] * 12 + [
            pltpu.VMEM_SHARED((N, DI), _f32),
            pltpu.VMEM_SHARED((N, 16), _f32),
        ],
        compiler_params=pltpu.CompilerParams(use_tc_tiling_on_sc=False),
    )
    return kfn(dst2d, src2d, pcfull, Vh)


def _sc_scatter_c_body(dst_hbm, pc_hbm, ac_hbm,
                       dst_v, pc_v,
                       isem0, isem1, isem2, isem3,
                       dsem0, dsem1, dsem2, dsem3,
                       ssem0, ssem1, ssem2, ssem3,
                       accc_sh):
    c = lax.axis_index("c")
    s = lax.axis_index("s")
    isems = (isem0, isem1, isem2, isem3)
    dsems = (dsem0, dsem1, dsem2, dsem3)
    ssems = (ssem0, ssem1, ssem2, ssem3)
    r0 = s * ROWS_PT

    def zrow(i, rc):
        for j4 in range(DI // 16):
            pc_v[0, i, pl.ds(j4 * 16, 16)] = jnp.zeros((16,), _f32)
        return rc
    lax.fori_loop(0, BV, zrow, 0)
    for t in range(ROWS_PT // BV):
        pltpu.sync_copy(pc_v.at[0], accc_sh.at[pl.ds(r0 + t * BV, BV)])
    plsc.subcore_barrier()

    base = (s * NC + c) * EPW

    def idx_cps(blk_i, k):
        roff = base // CH + blk_i
        return [pltpu.make_async_copy(dst_hbm.at[pl.ds(roff, 1)],
                                      dst_v.at[pl.ds(k, 1)], isems[k])]

    def data_cps(blk_i, k):
        off = base + blk_i * BV
        return [pltpu.make_async_copy(pc_hbm.at[pl.ds(off, BV), pl.ds(0, DI)],
                                      pc_v.at[k], dsems[k])]

    def sc_cps(k):
        return [pltpu.make_async_copy(pc_v.at[k], accc_sh.at[dst_v.at[k]], ssems[k])]

    for d in idx_cps(0, 0):
        d.start()
    for d in idx_cps(1, 1):
        d.start()
    for d in idx_cps(0, 0):
        d.wait()
    for d in data_cps(0, 0):
        d.start()

    def gbody(g, carry):
        for k in range(4):
            cur = g * 4 + k
            k1 = (k + 1) % 4
            k2 = (k + 2) % 4
            if k < 2:
                @pl.when(g > 0)
                def _(k2=k2):
                    for d in sc_cps(k2):
                        d.wait()
            else:
                for d in sc_cps(k2):
                    d.wait()
            if k < 2:
                for d in idx_cps(cur + 2, k2):
                    d.start()
            else:
                @pl.when(g < GV - 1)
                def _(cur=cur, k2=k2):
                    for d in idx_cps(cur + 2, k2):
                        d.start()
            if k < 3:
                for d in idx_cps(cur + 1, k1):
                    d.wait()
                for d in data_cps(cur + 1, k1):
                    d.start()
            else:
                @pl.when(g < GV - 1)
                def _(cur=cur, k1=k1):
                    for d in idx_cps(cur + 1, k1):
                        d.wait()
                    for d in data_cps(cur + 1, k1):
                        d.start()
            for d in data_cps(cur, k):
                d.wait()
            pltpu.async_copy(pc_v.at[k], accc_sh.at[dst_v.at[k]], ssems[k], add=True)
        return carry
    lax.fori_loop(0, GV, gbody, 0)
    for k in (2, 3):
        for d in sc_cps(k):
            d.wait()
    plsc.subcore_barrier()
    pltpu.sync_copy(accc_sh.at[pl.ds(r0, ROWS_PT)], ac_hbm.at[c, pl.ds(r0, ROWS_PT)])


def _sc_scatter_c(dst2d, pc):
    kfn = pl.kernel(
        _sc_scatter_c_body,
        out_type=jax.ShapeDtypeStruct((NC, N, DI), _f32),
        mesh=_mesh(),
        scratch_types=[
            pltpu.VMEM((4, CH), jnp.int32),
            pltpu.VMEM((4, BV, DI), _f32),
        ] + [pltpu.SemaphoreType.DMA] * 12 + [
            pltpu.VMEM_SHARED((N, DI), _f32),
        ],
        compiler_params=pltpu.CompilerParams(use_tc_tiling_on_sc=False),
    )
    return kfn(dst2d, pc)


def _comb_body(av_ref, ac_ref, ap_ref, bwm_ref, e16_ref, out_ref):
    w16 = 1.0 / (ap_ref[0] + ap_ref[1] + 1e-16)
    wexp = jnp.dot(w16, e16_ref[...], preferred_element_type=_f32)
    aggv = (av_ref[0] + av_ref[1]) * wexp
    aggc = (ac_ref[0] + ac_ref[1]) * wexp
    out_ref[...] = aggv + jnp.dot(aggc, bwm_ref[...], preferred_element_type=_f32)


def _tc_comb(av, ac, ap, BWm, E16):
    spec_acc = pl.BlockSpec((NC, BN, DI), lambda i: (0, i, 0))
    return pl.pallas_call(
        _comb_body,
        grid=(N // BN,),
        in_specs=[spec_acc, spec_acc,
                  pl.BlockSpec((NC, BN, 16), lambda i: (0, i, 0)),
                  pl.BlockSpec((DI, DI), lambda i: (0, 0)),
                  pl.BlockSpec((16, DI), lambda i: (0, 0))],
        out_specs=pl.BlockSpec((BN, DI), lambda i: (i, 0)),
        out_shape=jax.ShapeDtypeStruct((N, DI), _f32),
    )(av, ac, ap, BWm, E16)


def kernel(x, rrwp_index, rrwp_conn, WQ, WK, WV, WEw, WEb, bEb, WEo, bEo, Aw, BW):
    dst2d = rrwp_index[0].astype(jnp.int32).reshape(E // CH, CH)
    src2d = rrwp_index[1].astype(jnp.int32).reshape(E // CH, CH)

    eye = jnp.eye(HEADS, dtype=_f32)
    Awm = (Aw[:, :, 0].T[:, :, None] * eye[:, None, :]).reshape(DI, HEADS)
    E8 = jnp.kron(eye, jnp.ones((1, DIM), _f32))
    E16 = jnp.concatenate([E8, jnp.zeros((8, DI), _f32)], axis=0)
    I16 = jnp.concatenate([eye, jnp.zeros((8, 8), _f32)], axis=1)
    BWm = jnp.einsum('dhc,hg->hdgc', BW, eye).reshape(DI, DI)

    Qh, Kh, Vh = _tc_proj(x, WQ, WK, WV)
    msg = _sc_gather(dst2d, src2d, Qh, Kh)
    OeT, pcfull = _tc_edge(rrwp_conn, msg, WEw, WEb, bEb.reshape(1, DI),
                           WEo, bEo.reshape(1, DI), Awm, E8, I16)
    av, ap = _sc_scatter_v(dst2d, src2d, pcfull, Vh)
    ac = _sc_scatter_c(dst2d, pcfull)
    h_out = _tc_comb(av, ac, ap, BWm, E16)
    return (h_out, OeT.T)
